# baseline (speedup 1.0000x reference)
import jax
import jax.numpy as jnp
from jax import lax
from jax.experimental import pallas as pl
from jax.experimental.pallas import tpu as pltpu
from jax.experimental.pallas import tpu_sc as plsc

_NC = 2
_NS = 16
_L = 16
_NW = _NC * _NS
_B = 16384
_F = 32
_BPW = _B // _NW
_G = 4
_GR = 8
_TC = 128
_W = 8


def _body(user_hbm, item_hbm, uemb_hbm, iemb_hbm, out_hbm,
          vidx_u, vidx_i, tiles_u, tiles_v, out_v, sem):
    wid = lax.axis_index("s") * _NC + lax.axis_index("c")
    base = wid * _BPW
    lanes = lax.iota(jnp.int32, _L)

    pltpu.sync_copy(user_hbm.at[pl.ds(base, _BPW)], vidx_u)
    pltpu.sync_copy(item_hbm.at[pl.ds(base, _BPW)], vidx_i)

    def group(gp, carry):
        craw_u = vidx_u[pl.ds(gp * _L, _L)]
        craw_i = vidx_i[pl.ds(gp * _L, _L)]
        ccol_u = jnp.bitwise_and(craw_u, _TC - 1)
        ccol_i = jnp.bitwise_and(craw_i, _TC - 1)
        acc = jnp.zeros((_L,), jnp.float32)
        for w in range(_L // _W):
            copies = []
            for r in range(_W):
                tu = jnp.right_shift(craw_u[w * _W + r], 7)
                ti = jnp.right_shift(craw_i[w * _W + r], 7)
                for g in range(_G):
                    copies.append(pltpu.async_copy(
                        uemb_hbm.at[g, :, pl.ds(tu * _TC, _TC)],
                        tiles_u.at[r * _G + g], sem))
                    copies.append(pltpu.async_copy(
                        iemb_hbm.at[g, :, pl.ds(ti * _TC, _TC)],
                        tiles_v.at[r * _G + g], sem))
            for cp in copies:
                cp.wait()
            valid = (jnp.right_shift(lanes, 3) == w)
            slot0 = jnp.clip(lanes - w * _W, 0, _W - 1) * _G
            wacc = jnp.zeros((_L,), jnp.float32)
            for f in range(_F):
                rsub = jnp.full((_L,), f % _GR, jnp.int32)
                gu = plsc.load_gather(tiles_u, [slot0 + f // _GR, rsub, ccol_u])
                gv = plsc.load_gather(tiles_v, [slot0 + f // _GR, rsub, ccol_i])
                wacc = wacc + gu * gv
            acc = jnp.where(valid, wacc, acc)
        out_v[pl.ds(gp * _L, _L)] = 1.0 / (1.0 + jnp.exp(-acc))
        return carry

    lax.fori_loop(0, _BPW // _L, group, 0)

    pltpu.sync_copy(out_v, out_hbm.at[pl.ds(base, _BPW)])


def kernel(user, item, user_emb, item_emb):
    uemb3 = user_emb.T.reshape(_G, _GR, user_emb.shape[0])
    iemb3 = item_emb.T.reshape(_G, _GR, item_emb.shape[0])
    run = pl.kernel(
        _body,
        out_type=jax.ShapeDtypeStruct((_B,), jnp.float32),
        mesh=plsc.VectorSubcoreMesh(
            core_axis_name="c", subcore_axis_name="s",
            num_cores=_NC, num_subcores=_NS),
        scratch_types=[
            pltpu.VMEM((_BPW,), jnp.int32),
            pltpu.VMEM((_BPW,), jnp.int32),
            pltpu.VMEM((_W * _G, _GR, _TC), jnp.float32),
            pltpu.VMEM((_W * _G, _GR, _TC), jnp.float32),
            pltpu.VMEM((_BPW,), jnp.float32),
            pltpu.SemaphoreType.DMA,
        ],
        compiler_params=pltpu.CompilerParams(
            needs_layout_passes=False, use_tc_tiling_on_sc=True),
    )
    return run(user.astype(jnp.int32), item.astype(jnp.int32), uemb3, iemb3)


# speedup vs baseline: 1.1132x; 1.1132x over previous
import jax
import jax.numpy as jnp
from jax import lax
from jax.experimental import pallas as pl
from jax.experimental.pallas import tpu as pltpu
from jax.experimental.pallas import tpu_sc as plsc

_NC = 2
_NS = 16
_L = 16
_NW = _NC * _NS
_B = 16384
_F = 32
_BPW = _B // _NW
_G = 4
_GR = 8
_TC = 128
_W = 4


def _body(user_hbm, item_hbm, uemb_hbm, iemb_hbm, out_hbm,
          vidx_u, vidx_i, tiles_u, tiles_v, out_v, sem):
    wid = lax.axis_index("s") * _NC + lax.axis_index("c")
    base = wid * _BPW
    lanes = lax.iota(jnp.int32, _L)

    pltpu.sync_copy(user_hbm.at[pl.ds(base, _BPW)], vidx_u)
    pltpu.sync_copy(item_hbm.at[pl.ds(base, _BPW)], vidx_i)

    nq = _L // _W

    def group(gp, carry):
        craw_u = vidx_u[pl.ds(gp * _L, _L)]
        craw_i = vidx_i[pl.ds(gp * _L, _L)]
        ccol_u = jnp.bitwise_and(craw_u, _TC - 1)
        ccol_i = jnp.bitwise_and(craw_i, _TC - 1)

        def fire(q, s):
            copies = []
            for r in range(_W):
                tu = jnp.right_shift(craw_u[q * _W + r], 7)
                ti = jnp.right_shift(craw_i[q * _W + r], 7)
                for g in range(_G):
                    copies.append(pltpu.async_copy(
                        uemb_hbm.at[g, :, pl.ds(tu * _TC, _TC)],
                        tiles_u.at[(s * _W + r) * _G + g], sem))
                    copies.append(pltpu.async_copy(
                        iemb_hbm.at[g, :, pl.ds(ti * _TC, _TC)],
                        tiles_v.at[(s * _W + r) * _G + g], sem))
            return copies

        def compute(q, s, acc):
            valid = (jnp.right_shift(lanes, 2) == q)
            slot0 = (s * _W + jnp.clip(lanes - q * _W, 0, _W - 1)) * _G
            wacc = jnp.zeros((_L,), jnp.float32)
            for f in range(_F):
                rsub = jnp.full((_L,), f % _GR, jnp.int32)
                gu = plsc.load_gather(tiles_u, [slot0 + f // _GR, rsub, ccol_u])
                gv = plsc.load_gather(tiles_v, [slot0 + f // _GR, rsub, ccol_i])
                wacc = wacc + gu * gv
            return jnp.where(valid, wacc, acc)

        pend = {q: fire(q, q % 3) for q in range(min(3, nq))}
        acc = jnp.zeros((_L,), jnp.float32)
        for q in range(nq):
            for cp in pend.pop(q):
                cp.wait()
            acc = compute(q, q % 3, acc)
            if q + 3 < nq:
                pend[q + 3] = fire(q + 3, (q + 3) % 3)
        out_v[pl.ds(gp * _L, _L)] = 1.0 / (1.0 + jnp.exp(-acc))
        return carry

    lax.fori_loop(0, _BPW // _L, group, 0)

    pltpu.sync_copy(out_v, out_hbm.at[pl.ds(base, _BPW)])


def kernel(user, item, user_emb, item_emb):
    uemb3 = user_emb.T.reshape(_G, _GR, user_emb.shape[0])
    iemb3 = item_emb.T.reshape(_G, _GR, item_emb.shape[0])
    run = pl.kernel(
        _body,
        out_type=jax.ShapeDtypeStruct((_B,), jnp.float32),
        mesh=plsc.VectorSubcoreMesh(
            core_axis_name="c", subcore_axis_name="s",
            num_cores=_NC, num_subcores=_NS),
        scratch_types=[
            pltpu.VMEM((_BPW,), jnp.int32),
            pltpu.VMEM((_BPW,), jnp.int32),
            pltpu.VMEM((3 * _W * _G, _GR, _TC), jnp.float32),
            pltpu.VMEM((3 * _W * _G, _GR, _TC), jnp.float32),
            pltpu.VMEM((_BPW,), jnp.float32),
            pltpu.SemaphoreType.DMA,
        ],
        compiler_params=pltpu.CompilerParams(
            needs_layout_passes=False, use_tc_tiling_on_sc=True),
    )
    return run(user.astype(jnp.int32), item.astype(jnp.int32), uemb3, iemb3)


# speedup vs baseline: 1.2220x; 1.0977x over previous
import jax
import jax.numpy as jnp
from jax import lax
from jax.experimental import pallas as pl
from jax.experimental.pallas import tpu as pltpu
from jax.experimental.pallas import tpu_sc as plsc

_NC = 2
_NS = 16
_L = 16
_NW = _NC * _NS
_B = 16384
_F = 32
_BPW = _B // _NW
_G = 4
_GR = 8
_TC = 128
_W = 4


def _body(user_hbm, item_hbm, uemb_hbm, iemb_hbm, out_hbm,
          vidx_u, vidx_i, tiles_u, tiles_v, out_v, sem):
    wid = lax.axis_index("s") * _NC + lax.axis_index("c")
    base = wid * _BPW
    lanes = lax.iota(jnp.int32, _L)

    pltpu.sync_copy(user_hbm.at[pl.ds(base, _BPW)], vidx_u.at[pl.ds(0, _BPW)])
    pltpu.sync_copy(item_hbm.at[pl.ds(base, _BPW)], vidx_i.at[pl.ds(0, _BPW)])
    vidx_u[pl.ds(_BPW, _L)] = jnp.zeros((_L,), jnp.int32)
    vidx_i[pl.ds(_BPW, _L)] = jnp.zeros((_L,), jnp.int32)

    nq = _BPW // _W
    ring = 3
    gq = _L // _W

    def fire(q, s):
        cu = vidx_u[pl.ds(q * _W, _L)]
        ci = vidx_i[pl.ds(q * _W, _L)]
        for r in range(_W):
            tu = jnp.right_shift(cu[r], 7)
            ti = jnp.right_shift(ci[r], 7)
            for g in range(_G):
                pltpu.async_copy(
                    uemb_hbm.at[g, :, pl.ds(tu * _TC, _TC)],
                    tiles_u.at[(s * _W + r) * _G + g], sem)
                pltpu.async_copy(
                    iemb_hbm.at[g, :, pl.ds(ti * _TC, _TC)],
                    tiles_v.at[(s * _W + r) * _G + g], sem)

    for q in range(ring - 1):
        fire(q, q % ring)

    def quarter(q, acc):
        @pl.when(q + ring - 1 < nq)
        def _():
            fire(q + ring - 1, (q + ring - 1) % ring)
        for _ in range(2 * _W * _G):
            pltpu.make_async_copy(
                uemb_hbm.at[0, :, pl.ds(0, _TC)], tiles_u.at[0], sem).wait()
        s = q % ring
        qq = jnp.bitwise_and(q, gq - 1)
        g16 = jnp.right_shift(q, 2)
        craw_u = vidx_u[pl.ds(g16 * _L, _L)]
        craw_i = vidx_i[pl.ds(g16 * _L, _L)]
        ccol_u = jnp.bitwise_and(craw_u, _TC - 1)
        ccol_i = jnp.bitwise_and(craw_i, _TC - 1)
        valid = (jnp.right_shift(lanes, 2) == qq)
        slot0 = (s * _W + jnp.clip(lanes - qq * _W, 0, _W - 1)) * _G
        wacc = jnp.zeros((_L,), jnp.float32)
        for f in range(_F):
            rsub = jnp.full((_L,), f % _GR, jnp.int32)
            gu = plsc.load_gather(tiles_u, [slot0 + f // _GR, rsub, ccol_u])
            gv = plsc.load_gather(tiles_v, [slot0 + f // _GR, rsub, ccol_i])
            wacc = wacc + gu * gv
        res = jnp.where(valid, wacc, acc)

        @pl.when(qq == gq - 1)
        def _():
            out_v[pl.ds(g16 * _L, _L)] = 1.0 / (1.0 + jnp.exp(-res))
        return res

    lax.fori_loop(0, nq, quarter, jnp.zeros((_L,), jnp.float32))

    pltpu.sync_copy(out_v, out_hbm.at[pl.ds(base, _BPW)])


def kernel(user, item, user_emb, item_emb):
    uemb3 = user_emb.T.reshape(_G, _GR, user_emb.shape[0])
    iemb3 = item_emb.T.reshape(_G, _GR, item_emb.shape[0])
    run = pl.kernel(
        _body,
        out_type=jax.ShapeDtypeStruct((_B,), jnp.float32),
        mesh=plsc.VectorSubcoreMesh(
            core_axis_name="c", subcore_axis_name="s",
            num_cores=_NC, num_subcores=_NS),
        scratch_types=[
            pltpu.VMEM((_BPW + _L,), jnp.int32),
            pltpu.VMEM((_BPW + _L,), jnp.int32),
            pltpu.VMEM((3 * _W * _G, _GR, _TC), jnp.float32),
            pltpu.VMEM((3 * _W * _G, _GR, _TC), jnp.float32),
            pltpu.VMEM((_BPW,), jnp.float32),
            pltpu.SemaphoreType.DMA,
        ],
        compiler_params=pltpu.CompilerParams(
            needs_layout_passes=False, use_tc_tiling_on_sc=True),
    )
    return run(user.astype(jnp.int32), item.astype(jnp.int32), uemb3, iemb3)
